# baseline (device time: 37359 ns/iter reference)
import jax
import jax.numpy as jnp
from jax import lax
from jax.experimental import pallas as pl
from jax.experimental.pallas import tpu as pltpu

M = 2048
N = 1024
HALF_M = M // 2
HALF_N = N // 2
CHS = [128] * 7 + [64, 32, 16, 8, 8]
OFFS = [sum(CHS[:k]) for k in range(len(CHS))]
K = len(CHS)


def kernel(x):
    def body(x_ref, out_hbm, recv_ref, s_ref,
             sem_out, sem_sx, sem_rx, sem_sy, sem_ry):
        a = lax.axis_index("x")
        b = lax.axis_index("y")

        barrier_sem = pltpu.get_barrier_semaphore()
        pl.semaphore_signal(barrier_sem, inc=1, device_id=(1 - a, b),
                            device_id_type=pl.DeviceIdType.MESH)
        pl.semaphore_signal(barrier_sem, inc=1, device_id=(a, 1 - b),
                            device_id_type=pl.DeviceIdType.MESH)
        pl.semaphore_wait(barrier_sem, 2)

        base = b * HALF_M

        rdma_x = []
        for k in range(K):
            r = pltpu.make_async_remote_copy(
                src_ref=x_ref.at[0, pl.ds(base + OFFS[k], CHS[k]),
                                 pl.ds((1 - a) * HALF_N, HALF_N)],
                dst_ref=recv_ref.at[pl.ds(OFFS[k], CHS[k])],
                send_sem=sem_sx.at[k],
                recv_sem=sem_rx.at[k],
                device_id=(1 - a, b),
                device_id_type=pl.DeviceIdType.MESH,
            )
            r.start()
            rdma_x.append(r)

        rdma_y = []
        out_cp = []
        for k in range(K):
            lo, sz = OFFS[k], CHS[k]
            rows_k = pl.ds(base + lo, sz)
            rdma_x[k].wait_recv()
            s_ref[lo:lo + sz, :] = (
                x_ref[0, rows_k, pl.ds(a * HALF_N, HALF_N)]
                + recv_ref[lo:lo + sz, :]
            )
            c = pltpu.make_async_copy(
                s_ref.at[pl.ds(lo, sz)], out_hbm.at[rows_k, :], sem_out.at[k]
            )
            c.start()
            out_cp.append(c)
            r = pltpu.make_async_remote_copy(
                src_ref=s_ref.at[pl.ds(lo, sz)],
                dst_ref=out_hbm.at[rows_k, :],
                send_sem=sem_sy.at[k],
                recv_sem=sem_ry.at[k],
                device_id=(a, 1 - b),
                device_id_type=pl.DeviceIdType.MESH,
            )
            r.start()
            rdma_y.append(r)

        for k in range(K):
            rdma_x[k].wait_send()
            out_cp[k].wait()
            rdma_y[k].wait()

    return pl.pallas_call(
        body,
        out_shape=jax.ShapeDtypeStruct((M, HALF_N), jnp.float32),
        in_specs=[pl.BlockSpec(memory_space=pltpu.VMEM)],
        out_specs=pl.BlockSpec(memory_space=pl.ANY),
        scratch_shapes=[
            pltpu.VMEM((HALF_M, HALF_N), jnp.float32),
            pltpu.VMEM((HALF_M, HALF_N), jnp.float32),
            pltpu.SemaphoreType.DMA((K,)),
            pltpu.SemaphoreType.DMA((K,)),
            pltpu.SemaphoreType.DMA((K,)),
            pltpu.SemaphoreType.DMA((K,)),
            pltpu.SemaphoreType.DMA((K,)),
        ],
        compiler_params=pltpu.CompilerParams(collective_id=0),
    )(x)


# device time: 35703 ns/iter; 1.0464x vs baseline; 1.0464x over previous
import jax
import jax.numpy as jnp
from jax import lax
from jax.experimental import pallas as pl
from jax.experimental.pallas import tpu as pltpu

M = 2048
N = 1024
HALF_M = M // 2
HALF_N = N // 2
K = 32
CH = HALF_M // K


def kernel(x):
    def body(x_ref, out_ref, recv_ref, sem_sx, sem_rx, sem_sy, sem_ry):
        a = lax.axis_index("x")
        b = lax.axis_index("y")

        barrier_sem = pltpu.get_barrier_semaphore()
        pl.semaphore_signal(barrier_sem, inc=1, device_id=(1 - a, b),
                            device_id_type=pl.DeviceIdType.MESH)
        pl.semaphore_signal(barrier_sem, inc=1, device_id=(a, 1 - b),
                            device_id_type=pl.DeviceIdType.MESH)
        pl.semaphore_wait(barrier_sem, 2)

        base = b * HALF_M

        rdma_x = []
        for k in range(K):
            rows_k = pl.ds(base + k * CH, CH)
            r = pltpu.make_async_remote_copy(
                src_ref=x_ref.at[0, rows_k, pl.ds((1 - a) * HALF_N, HALF_N)],
                dst_ref=recv_ref.at[k],
                send_sem=sem_sx.at[k],
                recv_sem=sem_rx.at[k],
                device_id=(1 - a, b),
                device_id_type=pl.DeviceIdType.MESH,
            )
            r.start()
            rdma_x.append(r)

        rdma_y = []
        for k in range(K):
            rows_k = pl.ds(base + k * CH, CH)
            rdma_x[k].wait_recv()
            out_ref[rows_k, :] = (
                x_ref[0, rows_k, pl.ds(a * HALF_N, HALF_N)] + recv_ref[k]
            )
            r = pltpu.make_async_remote_copy(
                src_ref=out_ref.at[rows_k, :],
                dst_ref=out_ref.at[rows_k, :],
                send_sem=sem_sy.at[k],
                recv_sem=sem_ry.at[k],
                device_id=(a, 1 - b),
                device_id_type=pl.DeviceIdType.MESH,
            )
            r.start()
            rdma_y.append(r)

        for k in range(K):
            rdma_x[k].wait_send()
            rdma_y[k].wait()

    return pl.pallas_call(
        body,
        out_shape=jax.ShapeDtypeStruct((M, HALF_N), jnp.float32),
        in_specs=[pl.BlockSpec(memory_space=pltpu.VMEM)],
        out_specs=pl.BlockSpec(memory_space=pltpu.VMEM),
        scratch_shapes=[
            pltpu.VMEM((K, CH, HALF_N), jnp.float32),
            pltpu.SemaphoreType.DMA((K,)),
            pltpu.SemaphoreType.DMA((K,)),
            pltpu.SemaphoreType.DMA((K,)),
            pltpu.SemaphoreType.DMA((K,)),
        ],
        compiler_params=pltpu.CompilerParams(collective_id=0),
    )(x)
